# baseline (speedup 1.0000x reference)
import functools

import jax
import jax.numpy as jnp
from jax import lax
from jax.experimental import pallas as pl
from jax.experimental.pallas import tpu as pltpu
from jax.experimental.pallas import tpu_sc as plsc


def _sc_scatter(sidx3, w3, B, N):
    NN = N * N
    PER_CORE = 8 * NN
    SLICE = PER_CORE // 16
    mesh = plsc.VectorSubcoreMesh(core_axis_name="c", subcore_axis_name="s")

    @functools.partial(
        pl.kernel,
        mesh=mesh,
        out_type=jax.ShapeDtypeStruct((B * NN,), jnp.float32),
        scratch_types=[
            pltpu.VMEM((8192,), jnp.float32),
            pltpu.VMEM((2, 128), jnp.int32),
            pltpu.VMEM((2, 128), jnp.float32),
            pltpu.VMEM_SHARED((PER_CORE,), jnp.float32),
            pltpu.SemaphoreType.DMA,
        ],
    )
    def sc_kernel(sidx_hbm, w_hbm, a_hbm, zbuf, idx_v, w_v, a_sh, sem):
        c = lax.axis_index("c")
        s = lax.axis_index("s")
        wid = c * 16 + s

        ci = pltpu.async_copy(sidx_hbm.at[wid], idx_v, sem)
        cw = pltpu.async_copy(w_hbm.at[wid], w_v, sem)

        @pl.loop(0, 8192, step=16)
        def _(t):
            zbuf[pl.ds(t, 16)] = jnp.zeros((16,), jnp.float32)

        copies = [
            pltpu.async_copy(
                zbuf, a_sh.at[pl.ds(s * SLICE + t * 8192, 8192)], sem)
            for t in range(4)
        ]
        ci.wait()
        cw.wait()
        for cz in copies:
            cz.wait()
        plsc.subcore_barrier()

        for k in range(2):
            pltpu.sync_copy(w_v.at[k], a_sh.at[idx_v.at[k]], add=True)
        plsc.subcore_barrier()

        pltpu.sync_copy(
            a_sh.at[pl.ds(s * SLICE, SLICE)],
            a_hbm.at[pl.ds(c * PER_CORE + s * SLICE, SLICE)],
        )

    return sc_kernel(sidx3, w3)


_QB = 16


def _tc_body(d_ref, w_ref, p_ref, a_ref, out_ref):
    b = pl.program_id(0)
    nb = pl.num_programs(0)
    dsw = (3.0 * jnp.maximum(d_ref[...] - 1.0, 0.0)).astype(jnp.bfloat16)
    acc = jnp.float32(0.0)
    for q in range(_QB):
        p = p_ref[q].astype(jnp.bfloat16)
        n_half = p.shape[0] // 2
        t1 = lax.dot(p, dsw, preferred_element_type=jnp.float32)
        a0 = a_ref[q, 0].astype(jnp.bfloat16)
        a1 = a_ref[q, 1].astype(jnp.bfloat16)
        t2 = (lax.dot(a0, p[:n_half, :], preferred_element_type=jnp.float32)
              + lax.dot(a1, p[n_half:, :], preferred_element_type=jnp.float32))
        num = jnp.sum(t1 * t2)
        den = jnp.sum(w_ref[q, 0])
        acc += num / jnp.maximum(den, 1e-8)

    @pl.when(b == 0)
    def _():
        out_ref[0, 0] = 0.0

    out_ref[0, 0] += acc / (nb * _QB)


def _tc_fused(d_hw, w3d, Pb, A4):
    B, N, _ = Pb.shape
    E = w3d.shape[-1]
    return pl.pallas_call(
        _tc_body,
        grid=(B // _QB,),
        in_specs=[
            pl.BlockSpec((N, N), lambda b: (0, 0)),
            pl.BlockSpec((_QB, 1, E), lambda b: (b, 0, 0)),
            pl.BlockSpec((_QB, N, N), lambda b: (b, 0, 0)),
            pl.BlockSpec((_QB, 2, N, N // 2), lambda b: (b, 0, 0, 0)),
        ],
        out_specs=pl.BlockSpec((1, 1), lambda b: (0, 0),
                               memory_space=pltpu.SMEM),
        out_shape=jax.ShapeDtypeStruct((1, 1), jnp.float32),
    )(d_hw, w3d, Pb, A4)


def kernel(P, d_hw, circuit_edge_pairs, circuit_edge_weights):
    B, N, _ = P.shape
    _, E, _ = circuit_edge_pairs.shape
    NW = 32

    pairs = circuit_edge_pairs.astype(jnp.int32)
    i_idx = pairs[..., 0]
    j_idx = pairs[..., 1]
    lb = (jnp.arange(B, dtype=jnp.int32) % 8)[:, None]
    sidx = lb * (N * N) + ((j_idx >> 7) * N + i_idx) * 128 + (j_idx & 127)
    sidx3 = sidx.reshape(NW, (B * E) // NW // 128, 128)
    w3 = circuit_edge_weights.reshape(NW, (B * E) // NW // 128, 128)
    a_flat = _sc_scatter(sidx3, w3, B, N)
    A4 = a_flat.reshape(B, 2, N, N // 2)
    w3d = circuit_edge_weights.reshape(B, 1, E)
    out = _tc_fused(d_hw, w3d, P, A4)
    return out[0, 0]


# speedup vs baseline: 1.0205x; 1.0205x over previous
import functools

import jax
import jax.numpy as jnp
from jax import lax
from jax.experimental import pallas as pl
from jax.experimental.pallas import tpu as pltpu
from jax.experimental.pallas import tpu_sc as plsc


def _sc_scatter(sidx3, w3, B, N):
    NN = N * N
    PER_CORE = 8 * NN
    SLICE = PER_CORE // 16
    mesh = plsc.VectorSubcoreMesh(core_axis_name="c", subcore_axis_name="s")

    @functools.partial(
        pl.kernel,
        mesh=mesh,
        out_type=jax.ShapeDtypeStruct((B * NN,), jnp.float32),
        scratch_types=[
            pltpu.VMEM((8192,), jnp.float32),
            pltpu.VMEM((2, 128), jnp.int32),
            pltpu.VMEM((2, 128), jnp.float32),
            pltpu.VMEM_SHARED((PER_CORE,), jnp.float32),
            pltpu.SemaphoreType.DMA,
        ],
    )
    def sc_kernel(sidx_hbm, w_hbm, a_hbm, zbuf, idx_v, w_v, a_sh, sem):
        c = lax.axis_index("c")
        s = lax.axis_index("s")
        wid = c * 16 + s

        ci = pltpu.async_copy(sidx_hbm.at[wid], idx_v, sem)
        cw = pltpu.async_copy(w_hbm.at[wid], w_v, sem)

        @pl.loop(0, 8192, step=16)
        def _(t):
            zbuf[pl.ds(t, 16)] = jnp.zeros((16,), jnp.float32)

        copies = [
            pltpu.async_copy(
                zbuf, a_sh.at[pl.ds(s * SLICE + t * 8192, 8192)], sem)
            for t in range(4)
        ]
        ci.wait()
        cw.wait()
        for cz in copies:
            cz.wait()
        plsc.subcore_barrier()

        for k in range(2):
            pltpu.sync_copy(w_v.at[k], a_sh.at[idx_v.at[k]], add=True)
        plsc.subcore_barrier()

        pltpu.sync_copy(
            a_sh.at[pl.ds(s * SLICE, SLICE)],
            a_hbm.at[pl.ds(c * PER_CORE + s * SLICE, SLICE)],
        )

    return sc_kernel(sidx3, w3)


_QB = 8


def _tc_body(d_ref, w_ref, p_ref, a_ref, out_ref):
    b = pl.program_id(0)
    nb = pl.num_programs(0)
    dsw = (3.0 * jnp.maximum(d_ref[...] - 1.0, 0.0)).astype(jnp.bfloat16)
    acc = jnp.float32(0.0)
    for q in range(_QB):
        p = p_ref[q].astype(jnp.bfloat16)
        n_half = p.shape[0] // 2
        t1 = lax.dot(p, dsw, preferred_element_type=jnp.float32)
        a0 = a_ref[q, 0].astype(jnp.bfloat16)
        a1 = a_ref[q, 1].astype(jnp.bfloat16)
        t2 = (lax.dot(a0, p[:n_half, :], preferred_element_type=jnp.float32)
              + lax.dot(a1, p[n_half:, :], preferred_element_type=jnp.float32))
        num = jnp.sum(t1 * t2)
        den = jnp.sum(w_ref[q, 0])
        acc += num / jnp.maximum(den, 1e-8)

    @pl.when(b == 0)
    def _():
        out_ref[0, 0] = 0.0

    out_ref[0, 0] += acc / (nb * _QB)


def _tc_fused(d_hw, w3d, Pb, A4):
    B, N, _ = Pb.shape
    E = w3d.shape[-1]
    return pl.pallas_call(
        _tc_body,
        grid=(B // _QB,),
        in_specs=[
            pl.BlockSpec((N, N), lambda b: (0, 0)),
            pl.BlockSpec((_QB, 1, E), lambda b: (b, 0, 0)),
            pl.BlockSpec((_QB, N, N), lambda b: (b, 0, 0)),
            pl.BlockSpec((_QB, 2, N, N // 2), lambda b: (b, 0, 0, 0)),
        ],
        out_specs=pl.BlockSpec((1, 1), lambda b: (0, 0),
                               memory_space=pltpu.SMEM),
        out_shape=jax.ShapeDtypeStruct((1, 1), jnp.float32),
    )(d_hw, w3d, Pb, A4)


def kernel(P, d_hw, circuit_edge_pairs, circuit_edge_weights):
    B, N, _ = P.shape
    _, E, _ = circuit_edge_pairs.shape
    NW = 32

    pairs = circuit_edge_pairs.astype(jnp.int32)
    i_idx = pairs[..., 0]
    j_idx = pairs[..., 1]
    lb = (jnp.arange(B, dtype=jnp.int32) % 8)[:, None]
    sidx = lb * (N * N) + ((j_idx >> 7) * N + i_idx) * 128 + (j_idx & 127)
    sidx3 = sidx.reshape(NW, (B * E) // NW // 128, 128)
    w3 = circuit_edge_weights.reshape(NW, (B * E) // NW // 128, 128)
    a_flat = _sc_scatter(sidx3, w3, B, N)
    A4 = a_flat.reshape(B, 2, N, N // 2)
    w3d = circuit_edge_weights.reshape(B, 1, E)
    out = _tc_fused(d_hw, w3d, P, A4)
    return out[0, 0]
